# baseline (speedup 1.0000x reference)
import functools

import jax
import jax.numpy as jnp
from jax import lax
from jax.experimental import pallas as pl
from jax.experimental.pallas import tpu as pltpu
from jax.experimental.pallas import tpu_sc as plsc

N = 10000
E = 320000
C = 128
AW = 144
R = 1000
GRID = N // R
NT = 16
RPT = N // NT
B = 80


def _elu(x):
    return jnp.where(x > 0, x, jnp.exp(jnp.minimum(x, 0.0)) - 1.0)


def _prep(h, alog, heads, hd, tab_ref, init_ref, adv_ref, i):
    src_hd = hd % heads
    hh = h[:, src_hd * C:(src_hd + 1) * C]
    asrc = alog[:, 2 * src_hd]
    adst = alog[:, 2 * src_hd + 1]
    tab = jnp.concatenate(
        [hh, jnp.ones((R, 1), jnp.float32), asrc[:, None],
         jnp.zeros((R, AW - C - 2), jnp.float32)], axis=1)
    aself = asrc + adst
    wself = jnp.exp(jnp.where(aself > 0, aself, 0.2 * aself))
    if heads == 1 and hd == 1:
        init = jnp.zeros((R, AW), jnp.float32)
    else:
        init = wself[:, None] * tab
    tab_ref[hd] = tab
    init_ref[hd] = init
    adv_ref[0, hd, :] = adst


def _attn_cols(w_ref, as_ref, ad_ref, heads):
    cols = []
    for hd in range(heads):
        wh = w_ref[:, hd * C:(hd + 1) * C]
        cols.append(jnp.dot(wh, as_ref[hd, :][:, None],
                            preferred_element_type=jnp.float32))
        cols.append(jnp.dot(wh, ad_ref[hd, :][:, None],
                            preferred_element_type=jnp.float32))
    if heads == 1:
        cols = cols + cols
    return jnp.concatenate(cols, axis=1)


def _prep1_body(w_ref, as_ref, ad_ref, act_ref, tab_ref, init_ref, adv_ref):
    i = pl.program_id(0)
    act = act_ref[...]
    h = jnp.dot(act, w_ref[...], preferred_element_type=jnp.float32)
    alog = jnp.dot(act, _attn_cols(w_ref, as_ref, ad_ref, 2),
                   preferred_element_type=jnp.float32)
    for hd in range(2):
        _prep(h, alog, 2, hd, tab_ref, init_ref, adv_ref, i)


def _mid_body(heads_prev, heads, b_ref, w_ref, as_ref, ad_ref, acc0_ref, acc1_ref,
              tab_ref, init_ref, adv_ref):
    i = pl.program_id(0)
    acc0 = acc0_ref[...]
    acc1 = acc1_ref[...]
    if heads_prev == 2:
        act = jnp.concatenate(
            [acc0[:, :C] / acc0[:, C:C + 1], acc1[:, :C] / acc1[:, C:C + 1]], axis=1)
    else:
        s = acc0 + acc1
        act = s[:, :C] / s[:, C:C + 1]
    act = _elu(act + b_ref[0, :][None, :])
    h = jnp.dot(act, w_ref[...], preferred_element_type=jnp.float32)
    alog = jnp.dot(act, _attn_cols(w_ref, as_ref, ad_ref, heads),
                   preferred_element_type=jnp.float32)
    for hd in range(2):
        _prep(h, alog, heads, hd, tab_ref, init_ref, adv_ref, i)


def _final_body(b_ref, acc0_ref, acc1_ref, out_ref):
    s = acc0_ref[...] + acc1_ref[...]
    act = s[:, :C] / s[:, C:C + 1] + b_ref[0, :][None, :]
    out_ref[...] = _elu(act)


def _tc_prep1(x, W, a_s, a_d):
    return pl.pallas_call(
        _prep1_body,
        grid=(GRID,),
        in_specs=[
            pl.BlockSpec((128, 2 * C), lambda i: (0, 0)),
            pl.BlockSpec((2, C), lambda i: (0, 0)),
            pl.BlockSpec((2, C), lambda i: (0, 0)),
            pl.BlockSpec((R, 128), lambda i: (i, 0)),
        ],
        out_specs=[
            pl.BlockSpec((2, R, AW), lambda i: (0, i, 0)),
            pl.BlockSpec((2, R, AW), lambda i: (0, i, 0)),
            pl.BlockSpec((1, 2, R), lambda i: (i, 0, 0)),
        ],
        out_shape=[
            jax.ShapeDtypeStruct((2, N, AW), jnp.float32),
            jax.ShapeDtypeStruct((2, N, AW), jnp.float32),
            jax.ShapeDtypeStruct((GRID, 2, R), jnp.float32),
        ],
    )(W, a_s, a_d, x)


def _tc_mid(acc, b, W, a_s, a_d, heads_prev, heads):
    kin = heads_prev * C
    return pl.pallas_call(
        functools.partial(_mid_body, heads_prev, heads),
        grid=(GRID,),
        in_specs=[
            pl.BlockSpec((1, kin), lambda i: (0, 0)),
            pl.BlockSpec((kin, heads * C), lambda i: (0, 0)),
            pl.BlockSpec((heads, C), lambda i: (0, 0)),
            pl.BlockSpec((heads, C), lambda i: (0, 0)),
            pl.BlockSpec((R, AW), lambda i: (i, 0)),
            pl.BlockSpec((R, AW), lambda i: (GRID + i, 0)),
        ],
        out_specs=[
            pl.BlockSpec((2, R, AW), lambda i: (0, i, 0)),
            pl.BlockSpec((2, R, AW), lambda i: (0, i, 0)),
            pl.BlockSpec((1, 2, R), lambda i: (i, 0, 0)),
        ],
        out_shape=[
            jax.ShapeDtypeStruct((2, N, AW), jnp.float32),
            jax.ShapeDtypeStruct((2, N, AW), jnp.float32),
            jax.ShapeDtypeStruct((GRID, 2, R), jnp.float32),
        ],
    )(b.reshape(1, kin), W, a_s, a_d, acc, acc)


def _tc_final(acc, b):
    return pl.pallas_call(
        _final_body,
        grid=(GRID,),
        in_specs=[
            pl.BlockSpec((1, C), lambda i: (0, 0)),
            pl.BlockSpec((R, AW), lambda i: (i, 0)),
            pl.BlockSpec((R, AW), lambda i: (GRID + i, 0)),
        ],
        out_specs=pl.BlockSpec((R, C), lambda i: (i, 0)),
        out_shape=jax.ShapeDtypeStruct((N, C), jnp.float32),
    )(b.reshape(1, C), acc, acc)


def _make_sc_edge(edge_split):
    per_sc = E // 2 if edge_split else E
    per_tile = per_sc // NT
    BW = 40 if edge_split else B
    nwin = per_tile // BW
    assert nwin % 2 == 0
    mesh = plsc.VectorSubcoreMesh(core_axis_name="c", subcore_axis_name="s")

    @functools.partial(
        pl.kernel,
        out_type=jax.ShapeDtypeStruct((2 * N, AW), jnp.float32),
        mesh=mesh,
        compiler_params=pltpu.CompilerParams(use_tc_tiling_on_sc=False,
                                             needs_layout_passes=False),
        scratch_types=[
            pltpu.VMEM_SHARED((N, AW), jnp.float32),
            pltpu.VMEM((N,), jnp.float32),
            pltpu.VMEM((2, 3, BW), jnp.int32),
            pltpu.VMEM((2, BW), jnp.int32),
            pltpu.VMEM((2, BW, AW), jnp.float32),
            pltpu.VMEM((BW,), jnp.float32),
            pltpu.SemaphoreType.DMA,
            pltpu.SemaphoreType.DMA,
            pltpu.SemaphoreType.DMA,
            pltpu.SemaphoreType.DMA,
            pltpu.SemaphoreType.DMA,
            pltpu.SemaphoreType.DMA,
        ],
    )
    def sc_edge(tab_hbm, init_hbm, adst_hbm, idx3_hbm, out_hbm,
                acc, adst_t, idxb, dstb, rowb, wb,
                isem0, isem1, gsem0, gsem1, ssem0, ssem1):
        cid = lax.axis_index("c")
        sid = lax.axis_index("s")
        isem = (isem0, isem1)
        gsem = (gsem0, gsem1)
        ssem = (ssem0, ssem1)
        pltpu.sync_copy(adst_hbm.at[pl.ds(cid * N, N)], adst_t)
        r0 = sid * RPT
        pltpu.sync_copy(init_hbm.at[pl.ds(cid * N + r0, RPT)], acc.at[pl.ds(r0, RPT)])
        plsc.subcore_barrier()

        base = cid * per_sc + sid * per_tile

        def idx_start(g, p):
            pltpu.async_copy(idx3_hbm.at[:, pl.ds(base + g * BW, BW)],
                             idxb.at[p], isem[p])

        def idx_wait(p):
            pltpu.make_async_copy(idx3_hbm.at[:, pl.ds(base, BW)],
                                  idxb.at[p], isem[p]).wait()

        def gather_start(p):
            pltpu.async_copy(tab_hbm.at[idxb.at[p].at[0]], rowb.at[p], gsem[p])

        def gather_wait(p):
            pltpu.make_async_copy(tab_hbm.at[idxb.at[p].at[0]], rowb.at[p],
                                  gsem[p]).wait()

        def scat_start(p):
            pltpu.async_copy(rowb.at[p], acc.at[dstb.at[p]], ssem[p], add=True)

        def scat_wait(p):
            pltpu.make_async_copy(rowb.at[p], acc.at[dstb.at[p]],
                                  ssem[p]).wait()

        def compute(p):
            lanes = jnp.arange(16, dtype=jnp.int32)
            pfull = jnp.full((16,), p, jnp.int32)
            cfull = jnp.full((16,), C + 1, jnp.int32)
            @pl.loop(0, BW, step=16)
            def _alpha(j):
                d16 = idxb[p, 2, pl.ds(j, 16)]
                a_s = plsc.load_gather(rowb, [pfull, j + lanes, cfull])
                a = a_s + plsc.load_gather(adst_t, [d16])
                a = jnp.maximum(a, 0.2 * a)
                wb[pl.ds(j, 16)] = jnp.exp(a)
                dstb[p, pl.ds(j, 16)] = d16

        def scale(p):
            @pl.loop(0, BW)
            def _scale(e):
                w16 = plsc.load_gather(wb, [jnp.full((16,), e, jnp.int32)])
                for q in range(AW // 16):
                    rowb[p, e, pl.ds(q * 16, 16)] = w16 * rowb[p, e, pl.ds(q * 16, 16)]

        def window(g, p, np_, first):
            @pl.when(g + 1 < nwin)
            def _():
                idx_wait(np_)
                if not first:
                    scat_wait(np_)
                gather_start(np_)
            gather_wait(p)
            compute(p)
            @pl.when(g + 2 < nwin)
            def _():
                idx_start(g + 2, p)
            scale(p)
            scat_start(p)

        idx_start(0, 0)
        idx_start(1, 1)
        idx_wait(0)
        gather_start(0)
        window(0, 0, 1, True)
        window(1, 1, 0, False)

        @pl.loop(1, nwin // 2)
        def _h(h):
            g0 = 2 * h
            window(g0, 0, 1, False)
            window(g0 + 1, 1, 0, False)

        scat_wait(0)
        scat_wait(1)
        plsc.subcore_barrier()
        pltpu.sync_copy(acc.at[pl.ds(r0, RPT)], out_hbm.at[pl.ds(cid * N + r0, RPT)])

    return sc_edge


_sc_edge_heads = _make_sc_edge(False)
_sc_edge_split = _make_sc_edge(True)


def kernel(x, edge_index, W1, a_s1, a_d1, b1, W2, a_s2, a_d2, b2,
           W3, a_s3, a_d3, b3):
    src = edge_index[0].astype(jnp.int32)
    dst = edge_index[1].astype(jnp.int32)
    idx3h = jnp.stack([jnp.concatenate([src, src + N]),
                       jnp.concatenate([src, src]),
                       jnp.concatenate([dst, dst])])
    idx3s = jnp.stack([src, src, dst])

    def flat(a):
        return a.reshape(2 * N, AW)

    def flatv(a):
        return a.transpose(1, 0, 2).reshape(2 * N)

    tab, init, adv = _tc_prep1(x, W1, a_s1, a_d1)
    acc = _sc_edge_heads(flat(tab), flat(init), flatv(adv), idx3h)

    tab, init, adv = _tc_mid(acc, b1, W2, a_s2, a_d2, 2, 2)
    acc = _sc_edge_heads(flat(tab), flat(init), flatv(adv), idx3h)

    tab, init, adv = _tc_mid(acc, b2, W3, a_s3, a_d3, 2, 1)
    acc = _sc_edge_split(flat(tab), flat(init), flatv(adv), idx3s)

    return _tc_final(acc, b3)


# speedup vs baseline: 1.0003x; 1.0003x over previous
import functools

import jax
import jax.numpy as jnp
from jax import lax
from jax.experimental import pallas as pl
from jax.experimental.pallas import tpu as pltpu
from jax.experimental.pallas import tpu_sc as plsc

N = 10000
E = 320000
C = 128
AW = 144
R = 1000
GRID = N // R
NT = 16
RPT = N // NT
B = 80


def _elu(x):
    return jnp.where(x > 0, x, jnp.exp(jnp.minimum(x, 0.0)) - 1.0)


def _prep(h, heads, hd, as_ref, ad_ref, tab_ref, init_ref, adv_ref, i):
    src_hd = hd % heads
    hh = h[:, src_hd * C:(src_hd + 1) * C]
    a_s = as_ref[src_hd, :]
    a_d = ad_ref[src_hd, :]
    asrc = jnp.sum(hh * a_s[None, :], axis=1)
    adst = jnp.sum(hh * a_d[None, :], axis=1)
    tab = jnp.concatenate(
        [hh, jnp.ones((R, 1), jnp.float32), asrc[:, None],
         jnp.zeros((R, AW - C - 2), jnp.float32)], axis=1)
    aself = asrc + adst
    wself = jnp.exp(jnp.where(aself > 0, aself, 0.2 * aself))
    if heads == 1 and hd == 1:
        init = jnp.zeros((R, AW), jnp.float32)
    else:
        init = wself[:, None] * tab
    tab_ref[hd] = tab
    init_ref[hd] = init
    adv_ref[0, hd, :] = adst


def _prep1_body(w_ref, as_ref, ad_ref, act_ref, tab_ref, init_ref, adv_ref):
    i = pl.program_id(0)
    h = jnp.dot(act_ref[...], w_ref[...], preferred_element_type=jnp.float32)
    for hd in range(2):
        _prep(h, 2, hd, as_ref, ad_ref, tab_ref, init_ref, adv_ref, i)


def _mid_body(heads_prev, heads, b_ref, w_ref, as_ref, ad_ref, acc0_ref, acc1_ref,
              tab_ref, init_ref, adv_ref):
    i = pl.program_id(0)
    acc0 = acc0_ref[...]
    acc1 = acc1_ref[...]
    if heads_prev == 2:
        act = jnp.concatenate(
            [acc0[:, :C] / acc0[:, C:C + 1], acc1[:, :C] / acc1[:, C:C + 1]], axis=1)
    else:
        s = acc0 + acc1
        act = s[:, :C] / s[:, C:C + 1]
    act = _elu(act + b_ref[0, :][None, :])
    h = jnp.dot(act, w_ref[...], preferred_element_type=jnp.float32)
    for hd in range(2):
        _prep(h, heads, hd, as_ref, ad_ref, tab_ref, init_ref, adv_ref, i)


def _final_body(b_ref, acc0_ref, acc1_ref, out_ref):
    s = acc0_ref[...] + acc1_ref[...]
    act = s[:, :C] / s[:, C:C + 1] + b_ref[0, :][None, :]
    out_ref[...] = _elu(act)


def _tc_prep1(x, W, a_s, a_d):
    return pl.pallas_call(
        _prep1_body,
        grid=(GRID,),
        in_specs=[
            pl.BlockSpec((128, 2 * C), lambda i: (0, 0)),
            pl.BlockSpec((2, C), lambda i: (0, 0)),
            pl.BlockSpec((2, C), lambda i: (0, 0)),
            pl.BlockSpec((R, 128), lambda i: (i, 0)),
        ],
        out_specs=[
            pl.BlockSpec((2, R, AW), lambda i: (0, i, 0)),
            pl.BlockSpec((2, R, AW), lambda i: (0, i, 0)),
            pl.BlockSpec((1, 2, R), lambda i: (i, 0, 0)),
        ],
        out_shape=[
            jax.ShapeDtypeStruct((2, N, AW), jnp.float32),
            jax.ShapeDtypeStruct((2, N, AW), jnp.float32),
            jax.ShapeDtypeStruct((GRID, 2, R), jnp.float32),
        ],
    )(W, a_s, a_d, x)


def _tc_mid(acc, b, W, a_s, a_d, heads_prev, heads):
    kin = heads_prev * C
    return pl.pallas_call(
        functools.partial(_mid_body, heads_prev, heads),
        grid=(GRID,),
        in_specs=[
            pl.BlockSpec((1, kin), lambda i: (0, 0)),
            pl.BlockSpec((kin, heads * C), lambda i: (0, 0)),
            pl.BlockSpec((heads, C), lambda i: (0, 0)),
            pl.BlockSpec((heads, C), lambda i: (0, 0)),
            pl.BlockSpec((R, AW), lambda i: (i, 0)),
            pl.BlockSpec((R, AW), lambda i: (GRID + i, 0)),
        ],
        out_specs=[
            pl.BlockSpec((2, R, AW), lambda i: (0, i, 0)),
            pl.BlockSpec((2, R, AW), lambda i: (0, i, 0)),
            pl.BlockSpec((1, 2, R), lambda i: (i, 0, 0)),
        ],
        out_shape=[
            jax.ShapeDtypeStruct((2, N, AW), jnp.float32),
            jax.ShapeDtypeStruct((2, N, AW), jnp.float32),
            jax.ShapeDtypeStruct((GRID, 2, R), jnp.float32),
        ],
    )(b.reshape(1, kin), W, a_s, a_d, acc, acc)


def _tc_final(acc, b):
    return pl.pallas_call(
        _final_body,
        grid=(GRID,),
        in_specs=[
            pl.BlockSpec((1, C), lambda i: (0, 0)),
            pl.BlockSpec((R, AW), lambda i: (i, 0)),
            pl.BlockSpec((R, AW), lambda i: (GRID + i, 0)),
        ],
        out_specs=pl.BlockSpec((R, C), lambda i: (i, 0)),
        out_shape=jax.ShapeDtypeStruct((N, C), jnp.float32),
    )(b.reshape(1, C), acc, acc)


def _make_sc_edge(edge_split):
    per_sc = E // 2 if edge_split else E
    per_tile = per_sc // NT
    BW = 40 if edge_split else B
    nwin = per_tile // BW
    assert nwin % 2 == 0
    mesh = plsc.VectorSubcoreMesh(core_axis_name="c", subcore_axis_name="s")

    @functools.partial(
        pl.kernel,
        out_type=jax.ShapeDtypeStruct((2 * N, AW), jnp.float32),
        mesh=mesh,
        compiler_params=pltpu.CompilerParams(use_tc_tiling_on_sc=False,
                                             needs_layout_passes=False),
        scratch_types=[
            pltpu.VMEM_SHARED((N, AW), jnp.float32),
            pltpu.VMEM((N,), jnp.float32),
            pltpu.VMEM((2, 3, BW), jnp.int32),
            pltpu.VMEM((2, BW), jnp.int32),
            pltpu.VMEM((2, BW, AW), jnp.float32),
            pltpu.VMEM((BW,), jnp.float32),
            pltpu.SemaphoreType.DMA,
            pltpu.SemaphoreType.DMA,
            pltpu.SemaphoreType.DMA,
            pltpu.SemaphoreType.DMA,
            pltpu.SemaphoreType.DMA,
            pltpu.SemaphoreType.DMA,
        ],
    )
    def sc_edge(tab_hbm, init_hbm, adst_hbm, idx3_hbm, out_hbm,
                acc, adst_t, idxb, dstb, rowb, wb,
                isem0, isem1, gsem0, gsem1, ssem0, ssem1):
        cid = lax.axis_index("c")
        sid = lax.axis_index("s")
        isem = (isem0, isem1)
        gsem = (gsem0, gsem1)
        ssem = (ssem0, ssem1)
        pltpu.sync_copy(adst_hbm.at[pl.ds(cid * N, N)], adst_t)
        r0 = sid * RPT
        pltpu.sync_copy(init_hbm.at[pl.ds(cid * N + r0, RPT)], acc.at[pl.ds(r0, RPT)])
        plsc.subcore_barrier()

        base = cid * per_sc + sid * per_tile

        def idx_start(g, p):
            pltpu.async_copy(idx3_hbm.at[:, pl.ds(base + g * BW, BW)],
                             idxb.at[p], isem[p])

        def idx_wait(p):
            pltpu.make_async_copy(idx3_hbm.at[:, pl.ds(base, BW)],
                                  idxb.at[p], isem[p]).wait()

        def gather_start(p):
            pltpu.async_copy(tab_hbm.at[idxb.at[p].at[0]], rowb.at[p], gsem[p])

        def gather_wait(p):
            pltpu.make_async_copy(tab_hbm.at[idxb.at[p].at[0]], rowb.at[p],
                                  gsem[p]).wait()

        def scat_start(p):
            pltpu.async_copy(rowb.at[p], acc.at[dstb.at[p]], ssem[p], add=True)

        def scat_wait(p):
            pltpu.make_async_copy(rowb.at[p], acc.at[dstb.at[p]],
                                  ssem[p]).wait()

        def compute(p):
            lanes = jnp.arange(16, dtype=jnp.int32)
            pfull = jnp.full((16,), p, jnp.int32)
            cfull = jnp.full((16,), C + 1, jnp.int32)
            @pl.loop(0, BW, step=16)
            def _alpha(j):
                d16 = idxb[p, 2, pl.ds(j, 16)]
                a_s = plsc.load_gather(rowb, [pfull, j + lanes, cfull])
                a = a_s + plsc.load_gather(adst_t, [d16])
                a = jnp.maximum(a, 0.2 * a)
                wb[pl.ds(j, 16)] = jnp.exp(a)
                dstb[p, pl.ds(j, 16)] = d16

        def scale(p):
            @pl.loop(0, BW)
            def _scale(e):
                w16 = plsc.load_gather(wb, [jnp.full((16,), e, jnp.int32)])
                for q in range(AW // 16):
                    rowb[p, e, pl.ds(q * 16, 16)] = w16 * rowb[p, e, pl.ds(q * 16, 16)]

        def window(g, p, np_, first):
            @pl.when(g + 1 < nwin)
            def _():
                idx_wait(np_)
                if not first:
                    scat_wait(np_)
                gather_start(np_)
            gather_wait(p)
            compute(p)
            @pl.when(g + 2 < nwin)
            def _():
                idx_start(g + 2, p)
            scale(p)
            scat_start(p)

        idx_start(0, 0)
        idx_start(1, 1)
        idx_wait(0)
        gather_start(0)
        window(0, 0, 1, True)
        window(1, 1, 0, False)

        @pl.loop(1, nwin // 2)
        def _h(h):
            g0 = 2 * h
            window(g0, 0, 1, False)
            window(g0 + 1, 1, 0, False)

        scat_wait(0)
        scat_wait(1)
        plsc.subcore_barrier()
        pltpu.sync_copy(acc.at[pl.ds(r0, RPT)], out_hbm.at[pl.ds(cid * N + r0, RPT)])

    return sc_edge


_sc_edge_heads = _make_sc_edge(False)
_sc_edge_split = _make_sc_edge(True)


def kernel(x, edge_index, W1, a_s1, a_d1, b1, W2, a_s2, a_d2, b2,
           W3, a_s3, a_d3, b3):
    src = edge_index[0].astype(jnp.int32)
    dst = edge_index[1].astype(jnp.int32)
    idx3h = jnp.stack([jnp.concatenate([src, src + N]),
                       jnp.concatenate([src, src]),
                       jnp.concatenate([dst, dst])])
    idx3s = jnp.stack([src, src, dst])

    def flat(a):
        return a.reshape(2 * N, AW)

    def flatv(a):
        return a.transpose(1, 0, 2).reshape(2 * N)

    tab, init, adv = _tc_prep1(x, W1, a_s1, a_d1)
    acc = _sc_edge_heads(flat(tab), flat(init), flatv(adv), idx3h)

    tab, init, adv = _tc_mid(acc, b1, W2, a_s2, a_d2, 2, 2)
    acc = _sc_edge_heads(flat(tab), flat(init), flatv(adv), idx3h)

    tab, init, adv = _tc_mid(acc, b2, W3, a_s3, a_d3, 2, 1)
    acc = _sc_edge_split(flat(tab), flat(init), flatv(adv), idx3s)

    return _tc_final(acc, b3)


# speedup vs baseline: 1.0065x; 1.0062x over previous
import functools

import jax
import jax.numpy as jnp
from jax import lax
from jax.experimental import pallas as pl
from jax.experimental.pallas import tpu as pltpu
from jax.experimental.pallas import tpu_sc as plsc

N = 10000
E = 320000
C = 128
AW = 144
R = 1000
GRID = N // R
NT = 16
RPT = N // NT
B = 80


def _elu(x):
    return jnp.where(x > 0, x, jnp.exp(jnp.minimum(x, 0.0)) - 1.0)


def _prep(h, heads, hd, as_ref, ad_ref, tab_ref, init_ref, adv_ref, i):
    src_hd = hd % heads
    hh = h[:, src_hd * C:(src_hd + 1) * C]
    a_s = as_ref[src_hd, :]
    a_d = ad_ref[src_hd, :]
    asrc = jnp.sum(hh * a_s[None, :], axis=1)
    adst = jnp.sum(hh * a_d[None, :], axis=1)
    tab = jnp.concatenate(
        [hh, jnp.ones((R, 1), jnp.float32), asrc[:, None],
         jnp.zeros((R, AW - C - 2), jnp.float32)], axis=1)
    aself = asrc + adst
    wself = jnp.exp(jnp.where(aself > 0, aself, 0.2 * aself))
    if heads == 1 and hd == 1:
        init = jnp.zeros((R, AW), jnp.float32)
    else:
        init = wself[:, None] * tab
    tab_ref[hd] = tab
    init_ref[hd] = init
    adv_ref[0, hd, :] = adst


def _prep1_body(w_ref, as_ref, ad_ref, act_ref, tab_ref, init_ref, adv_ref):
    i = pl.program_id(0)
    h = jnp.dot(act_ref[...], w_ref[...], preferred_element_type=jnp.float32)
    for hd in range(2):
        _prep(h, 2, hd, as_ref, ad_ref, tab_ref, init_ref, adv_ref, i)


def _mid_body(heads_prev, heads, b_ref, w_ref, as_ref, ad_ref, acc0_ref, acc1_ref,
              tab_ref, init_ref, adv_ref):
    i = pl.program_id(0)
    acc0 = acc0_ref[...]
    acc1 = acc1_ref[...]
    if heads_prev == 2:
        act = jnp.concatenate(
            [acc0[:, :C] / acc0[:, C:C + 1], acc1[:, :C] / acc1[:, C:C + 1]], axis=1)
    else:
        s = acc0 + acc1
        act = s[:, :C] / s[:, C:C + 1]
    act = _elu(act + b_ref[0, :][None, :])
    h = jnp.dot(act, w_ref[...], preferred_element_type=jnp.float32)
    for hd in range(2):
        _prep(h, heads, hd, as_ref, ad_ref, tab_ref, init_ref, adv_ref, i)


def _final_body(b_ref, acc0_ref, acc1_ref, out_ref):
    s = acc0_ref[...] + acc1_ref[...]
    act = s[:, :C] / s[:, C:C + 1] + b_ref[0, :][None, :]
    out_ref[...] = _elu(act)


def _tc_prep1(x, W, a_s, a_d):
    return pl.pallas_call(
        _prep1_body,
        grid=(GRID,),
        in_specs=[
            pl.BlockSpec((128, 2 * C), lambda i: (0, 0)),
            pl.BlockSpec((2, C), lambda i: (0, 0)),
            pl.BlockSpec((2, C), lambda i: (0, 0)),
            pl.BlockSpec((R, 128), lambda i: (i, 0)),
        ],
        out_specs=[
            pl.BlockSpec((2, R, AW), lambda i: (0, i, 0)),
            pl.BlockSpec((2, R, AW), lambda i: (0, i, 0)),
            pl.BlockSpec((1, 2, R), lambda i: (i, 0, 0)),
        ],
        out_shape=[
            jax.ShapeDtypeStruct((2, N, AW), jnp.float32),
            jax.ShapeDtypeStruct((2, N, AW), jnp.float32),
            jax.ShapeDtypeStruct((GRID, 2, R), jnp.float32),
        ],
    )(W, a_s, a_d, x)


def _tc_mid(acc, b, W, a_s, a_d, heads_prev, heads):
    kin = heads_prev * C
    return pl.pallas_call(
        functools.partial(_mid_body, heads_prev, heads),
        grid=(GRID,),
        in_specs=[
            pl.BlockSpec((1, kin), lambda i: (0, 0)),
            pl.BlockSpec((kin, heads * C), lambda i: (0, 0)),
            pl.BlockSpec((heads, C), lambda i: (0, 0)),
            pl.BlockSpec((heads, C), lambda i: (0, 0)),
            pl.BlockSpec((R, AW), lambda i: (i, 0)),
            pl.BlockSpec((R, AW), lambda i: (GRID + i, 0)),
        ],
        out_specs=[
            pl.BlockSpec((2, R, AW), lambda i: (0, i, 0)),
            pl.BlockSpec((2, R, AW), lambda i: (0, i, 0)),
            pl.BlockSpec((1, 2, R), lambda i: (i, 0, 0)),
        ],
        out_shape=[
            jax.ShapeDtypeStruct((2, N, AW), jnp.float32),
            jax.ShapeDtypeStruct((2, N, AW), jnp.float32),
            jax.ShapeDtypeStruct((GRID, 2, R), jnp.float32),
        ],
    )(b.reshape(1, kin), W, a_s, a_d, acc, acc)


def _tc_final(acc, b):
    return pl.pallas_call(
        _final_body,
        grid=(GRID,),
        in_specs=[
            pl.BlockSpec((1, C), lambda i: (0, 0)),
            pl.BlockSpec((R, AW), lambda i: (i, 0)),
            pl.BlockSpec((R, AW), lambda i: (GRID + i, 0)),
        ],
        out_specs=pl.BlockSpec((R, C), lambda i: (i, 0)),
        out_shape=jax.ShapeDtypeStruct((N, C), jnp.float32),
    )(b.reshape(1, C), acc, acc)


def _make_sc_edge(edge_split):
    per_sc = E // 2 if edge_split else E
    per_tile = per_sc // NT
    BW = 40 if edge_split else B
    nwin = per_tile // BW
    assert nwin % 2 == 0
    mesh = plsc.VectorSubcoreMesh(core_axis_name="c", subcore_axis_name="s")

    @functools.partial(
        pl.kernel,
        out_type=jax.ShapeDtypeStruct((2 * N, AW), jnp.float32),
        mesh=mesh,
        compiler_params=pltpu.CompilerParams(use_tc_tiling_on_sc=False,
                                             needs_layout_passes=False),
        scratch_types=[
            pltpu.VMEM_SHARED((N, AW), jnp.float32),
            pltpu.VMEM((N,), jnp.float32),
            pltpu.VMEM((2, 2, BW), jnp.int32),
            pltpu.VMEM((2, BW), jnp.int32),
            pltpu.VMEM((2, BW, AW), jnp.float32),
            pltpu.VMEM((BW,), jnp.float32),
            pltpu.SemaphoreType.DMA,
            pltpu.SemaphoreType.DMA,
            pltpu.SemaphoreType.DMA,
            pltpu.SemaphoreType.DMA,
            pltpu.SemaphoreType.DMA,
            pltpu.SemaphoreType.DMA,
        ],
    )
    def sc_edge(tab_hbm, init_hbm, adst_hbm, idx3_hbm, out_hbm,
                acc, adst_t, idxb, dstb, rowb, wb,
                isem0, isem1, gsem0, gsem1, ssem0, ssem1):
        cid = lax.axis_index("c")
        sid = lax.axis_index("s")
        isem = (isem0, isem1)
        gsem = (gsem0, gsem1)
        ssem = (ssem0, ssem1)
        pltpu.sync_copy(adst_hbm.at[pl.ds(cid * N, N)], adst_t)
        r0 = sid * RPT
        pltpu.sync_copy(init_hbm.at[pl.ds(cid * N + r0, RPT)], acc.at[pl.ds(r0, RPT)])
        plsc.subcore_barrier()

        base = cid * per_sc + sid * per_tile

        def idx_start(g, p):
            pltpu.async_copy(idx3_hbm.at[:, pl.ds(base + g * BW, BW)],
                             idxb.at[p], isem[p])

        def idx_wait(p):
            pltpu.make_async_copy(idx3_hbm.at[:, pl.ds(base, BW)],
                                  idxb.at[p], isem[p]).wait()

        def gather_start(p):
            pltpu.async_copy(tab_hbm.at[idxb.at[p].at[0]], rowb.at[p], gsem[p])

        def gather_wait(p):
            pltpu.make_async_copy(tab_hbm.at[idxb.at[p].at[0]], rowb.at[p],
                                  gsem[p]).wait()

        def scat_start(p):
            pltpu.async_copy(rowb.at[p], acc.at[dstb.at[p]], ssem[p], add=True)

        def scat_wait(p):
            pltpu.make_async_copy(rowb.at[p], acc.at[dstb.at[p]],
                                  ssem[p]).wait()

        def compute(p):
            lanes = jnp.arange(16, dtype=jnp.int32)
            pfull = jnp.full((16,), p, jnp.int32)
            cfull = jnp.full((16,), C + 1, jnp.int32)
            @pl.loop(0, BW, step=16)
            def _alpha(j):
                d16 = idxb[p, 1, pl.ds(j, 16)]
                a_s = plsc.load_gather(rowb, [pfull, j + lanes, cfull])
                a = a_s + plsc.load_gather(adst_t, [d16])
                a = jnp.maximum(a, 0.2 * a)
                wb[pl.ds(j, 16)] = jnp.exp(a)
                dstb[p, pl.ds(j, 16)] = d16

        def scale(p):
            @pl.loop(0, BW)
            def _scale(e):
                w16 = plsc.load_gather(wb, [jnp.full((16,), e, jnp.int32)])
                for q in range(AW // 16):
                    rowb[p, e, pl.ds(q * 16, 16)] = w16 * rowb[p, e, pl.ds(q * 16, 16)]

        def window(g, p, np_, first):
            @pl.when(g + 1 < nwin)
            def _():
                idx_wait(np_)
                if not first:
                    scat_wait(np_)
                gather_start(np_)
            gather_wait(p)
            compute(p)
            @pl.when(g + 2 < nwin)
            def _():
                idx_start(g + 2, p)
            scale(p)
            scat_start(p)

        idx_start(0, 0)
        idx_start(1, 1)
        idx_wait(0)
        gather_start(0)
        window(0, 0, 1, True)
        window(1, 1, 0, False)

        @pl.loop(1, nwin // 2)
        def _h(h):
            g0 = 2 * h
            window(g0, 0, 1, False)
            window(g0 + 1, 1, 0, False)

        scat_wait(0)
        scat_wait(1)
        plsc.subcore_barrier()
        pltpu.sync_copy(acc.at[pl.ds(r0, RPT)], out_hbm.at[pl.ds(cid * N + r0, RPT)])

    return sc_edge


_sc_edge_heads = _make_sc_edge(False)
_sc_edge_split = _make_sc_edge(True)


def kernel(x, edge_index, W1, a_s1, a_d1, b1, W2, a_s2, a_d2, b2,
           W3, a_s3, a_d3, b3):
    src = edge_index[0].astype(jnp.int32)
    dst = edge_index[1].astype(jnp.int32)
    idx3h = jnp.stack([jnp.concatenate([src, src + N]),
                       jnp.concatenate([dst, dst])])
    idx3s = jnp.stack([src, dst])

    def flat(a):
        return a.reshape(2 * N, AW)

    def flatv(a):
        return a.transpose(1, 0, 2).reshape(2 * N)

    tab, init, adv = _tc_prep1(x, W1, a_s1, a_d1)
    acc = _sc_edge_heads(flat(tab), flat(init), flatv(adv), idx3h)

    tab, init, adv = _tc_mid(acc, b1, W2, a_s2, a_d2, 2, 2)
    acc = _sc_edge_heads(flat(tab), flat(init), flatv(adv), idx3h)

    tab, init, adv = _tc_mid(acc, b2, W3, a_s3, a_d3, 2, 1)
    acc = _sc_edge_split(flat(tab), flat(init), flatv(adv), idx3s)

    return _tc_final(acc, b3)
